# baseline (speedup 1.0000x reference)
import functools

import jax
import jax.numpy as jnp
from jax import lax
from jax.experimental import pallas as pl
from jax.experimental.pallas import tpu as pltpu
from jax.experimental.pallas import tpu_sc as plsc

E = 8
TOPK = 2
R = 8
DIM = 2048
ALPHA = 16
SCALING = ALPHA / R
ER = E * R

BN = 256

NC = 2
NS = 16
NW = NC * NS
L = 16


def _logits_body(x_ref, wgt_ref, bg_ref, logits_t_ref):
    xb = x_ref[...]
    logits = (
        jnp.dot(xb, wgt_ref[...], preferred_element_type=jnp.float32)
        + bg_ref[...]
    )
    logits_t_ref[...] = logits.T


def _act_body(x_ref, w1t_ref, act_ref):
    xb = x_ref[...]
    h = jnp.dot(xb, w1t_ref[...], preferred_element_type=jnp.float32)
    act_ref[...] = h * 0.5 * (1.0 + lax.erf(h * (2.0 ** -0.5)))


def _combine_body(gates_t_ref, act_ref, expand_ref, w2f_ref, out_ref):
    gr = jnp.dot(
        gates_t_ref[...].T,
        expand_ref[...],
        preferred_element_type=jnp.float32,
    )
    out_ref[...] = jnp.dot(
        gr * act_ref[...], w2f_ref[...], preferred_element_type=jnp.float32
    )


def _route_body(logits_t_hbm, gates_t_hbm, lg_v, gt_v, sem, tpw, n):
    wid = lax.axis_index("s") * NC + lax.axis_index("c")
    base = wid * tpw
    copies = [
        pltpu.async_copy(
            logits_t_hbm.at[pl.ds(e * n + base, tpw)],
            lg_v.at[pl.ds(e * tpw, tpw)],
            sem,
        )
        for e in range(E)
    ]
    for c in copies:
        c.wait()
    for g in range(tpw // L):
        ls = [lg_v[pl.ds(e * tpw + g * L, L)] for e in range(E)]
        m1 = ls[0]
        for e in range(1, E):
            m1 = jnp.maximum(m1, ls[e])
        i1 = jnp.full((L,), E, jnp.int32)
        for e in range(E - 1, -1, -1):
            i1 = jnp.where(ls[e] == m1, jnp.int32(e), i1)
        m2 = None
        for e in range(E):
            le = jnp.where(i1 == e, jnp.float32(-jnp.inf), ls[e])
            m2 = le if m2 is None else jnp.maximum(m2, le)
        i2 = jnp.full((L,), E, jnp.int32)
        for e in range(E - 1, -1, -1):
            cand = jnp.logical_and(ls[e] == m2, i1 != e)
            i2 = jnp.where(cand, jnp.int32(e), i2)
        g1 = 1.0 / (1.0 + jnp.exp(m2 - m1))
        g2 = 1.0 - g1
        for e in range(E):
            ge = jnp.where(
                i1 == e, g1, jnp.where(i2 == e, g2, jnp.float32(0.0))
            )
            gt_v[pl.ds(e * tpw + g * L, L)] = ge
    out_copies = [
        pltpu.async_copy(
            gt_v.at[pl.ds(e * tpw, tpw)],
            gates_t_hbm.at[pl.ds(e * n + base, tpw)],
            sem,
        )
        for e in range(E)
    ]
    for c in out_copies:
        c.wait()


def _route(logits_t_flat, n):
    tpw = n // NW
    mesh = plsc.VectorSubcoreMesh(core_axis_name="c", subcore_axis_name="s")
    body = functools.partial(_route_body, tpw=tpw, n=n)
    return pl.kernel(
        body,
        out_type=jax.ShapeDtypeStruct((E * n,), jnp.float32),
        mesh=mesh,
        scratch_types=[
            pltpu.VMEM((E * tpw,), jnp.float32),
            pltpu.VMEM((E * tpw,), jnp.float32),
            pltpu.SemaphoreType.DMA,
        ],
        name="moe_route_sc",
    )(logits_t_flat)


def kernel(x, Wg, bg, W1, W2):
    orig_shape = x.shape
    x_flat = x.reshape(-1, x.shape[-1])
    n = x_flat.shape[0]

    wgt = Wg.T
    w1t = W1.reshape(ER, DIM).T
    w2f = (W2 * SCALING).transpose(0, 2, 1).reshape(ER, DIM)
    bg2 = bg.reshape(1, E)
    expand = jnp.repeat(jnp.eye(E, dtype=jnp.float32), R, axis=1)

    grid = (n // BN,)
    logits_t = pl.pallas_call(
        _logits_body,
        grid=grid,
        in_specs=[
            pl.BlockSpec((BN, DIM), lambda i: (i, 0)),
            pl.BlockSpec((DIM, E), lambda i: (0, 0)),
            pl.BlockSpec((1, E), lambda i: (0, 0)),
        ],
        out_specs=pl.BlockSpec((E, BN), lambda i: (0, i)),
        out_shape=jax.ShapeDtypeStruct((E, n), jnp.float32),
        name="moe_logits_tc",
    )(x_flat, wgt, bg2)

    gates_t = _route(logits_t.reshape(E * n), n).reshape(E, n)

    act = pl.pallas_call(
        _act_body,
        grid=grid,
        in_specs=[
            pl.BlockSpec((BN, DIM), lambda i: (i, 0)),
            pl.BlockSpec((DIM, ER), lambda i: (0, 0)),
        ],
        out_specs=pl.BlockSpec((BN, ER), lambda i: (i, 0)),
        out_shape=jax.ShapeDtypeStruct((n, ER), jnp.float32),
        name="moe_act_tc",
    )(x_flat, w1t)

    out = pl.pallas_call(
        _combine_body,
        grid=grid,
        in_specs=[
            pl.BlockSpec((E, BN), lambda i: (0, i)),
            pl.BlockSpec((BN, ER), lambda i: (i, 0)),
            pl.BlockSpec((E, ER), lambda i: (0, 0)),
            pl.BlockSpec((ER, DIM), lambda i: (0, 0)),
        ],
        out_specs=pl.BlockSpec((BN, DIM), lambda i: (i, 0)),
        out_shape=jax.ShapeDtypeStruct((n, DIM), jnp.float32),
        name="moe_combine_tc",
    )(gates_t, act, expand, w2f)

    return out.reshape(orig_shape)


# speedup vs baseline: 1.6843x; 1.6843x over previous
import functools

import jax
import jax.numpy as jnp
from jax import lax
from jax.experimental import pallas as pl
from jax.experimental.pallas import tpu as pltpu
from jax.experimental.pallas import tpu_sc as plsc

E = 8
TOPK = 2
R = 8
DIM = 2048
ALPHA = 16
SCALING = ALPHA / R
ER = E * R

BN = 256

NC = 2
NS = 16
NW = NC * NS
L = 16


def _fwd_body(x_ref, wgt_ref, bg_ref, w1t_ref, logits_t_ref, act_ref):
    xb = x_ref[...]
    logits = (
        jnp.dot(xb, wgt_ref[...], preferred_element_type=jnp.float32)
        + bg_ref[...]
    )
    ii = lax.broadcasted_iota(jnp.int32, logits.shape, 1)
    m1 = jnp.max(logits, 1, keepdims=True)
    i1 = jnp.min(jnp.where(logits == m1, ii, E), 1, keepdims=True)
    m2 = jnp.max(jnp.where(ii == i1, -jnp.inf, logits), 1, keepdims=True)
    i2 = jnp.min(
        jnp.where((logits == m2) & (ii != i1), ii, E), 1, keepdims=True
    )
    g1 = 1.0 / (1.0 + jnp.exp(m2 - m1))
    g2 = 1.0 - g1
    gates = jnp.where(ii == i1, g1, jnp.where(ii == i2, g2, 0.0))
    logits_t_ref[...] = gates.T
    h = jnp.dot(xb, w1t_ref[...], preferred_element_type=jnp.float32)
    act_ref[...] = h * 0.5 * (1.0 + lax.erf(h * (2.0 ** -0.5)))


def _combine_body(gates_t_ref, act_ref, expand_ref, w2f_ref, out_ref):
    gr = jnp.dot(
        gates_t_ref[...].T,
        expand_ref[...],
        preferred_element_type=jnp.float32,
    )
    out_ref[...] = jnp.dot(
        gr * act_ref[...], w2f_ref[...], preferred_element_type=jnp.float32
    )


def _route_body(logits_t_hbm, gates_t_hbm, lg_v, gt_v, sem, tpw, n):
    wid = lax.axis_index("s") * NC + lax.axis_index("c")
    base = wid * tpw
    copies = [
        pltpu.async_copy(
            logits_t_hbm.at[pl.ds(e * n + base, tpw)],
            lg_v.at[pl.ds(e * tpw, tpw)],
            sem,
        )
        for e in range(E)
    ]
    for c in copies:
        c.wait()
    for g in range(tpw // L):
        ls = [lg_v[pl.ds(e * tpw + g * L, L)] for e in range(E)]
        m1 = ls[0]
        for e in range(1, E):
            m1 = jnp.maximum(m1, ls[e])
        i1 = jnp.full((L,), E, jnp.int32)
        for e in range(E - 1, -1, -1):
            i1 = jnp.where(ls[e] == m1, jnp.int32(e), i1)
        m2 = None
        for e in range(E):
            le = jnp.where(i1 == e, jnp.float32(-jnp.inf), ls[e])
            m2 = le if m2 is None else jnp.maximum(m2, le)
        i2 = jnp.full((L,), E, jnp.int32)
        for e in range(E - 1, -1, -1):
            cand = jnp.logical_and(ls[e] == m2, i1 != e)
            i2 = jnp.where(cand, jnp.int32(e), i2)
        g1 = 1.0 / (1.0 + jnp.exp(m2 - m1))
        g2 = 1.0 - g1
        for e in range(E):
            ge = jnp.where(
                i1 == e, g1, jnp.where(i2 == e, g2, jnp.float32(0.0))
            )
            gt_v[pl.ds(e * tpw + g * L, L)] = ge
    out_copies = [
        pltpu.async_copy(
            gt_v.at[pl.ds(e * tpw, tpw)],
            gates_t_hbm.at[pl.ds(e * n + base, tpw)],
            sem,
        )
        for e in range(E)
    ]
    for c in out_copies:
        c.wait()


def _route(logits_t_flat, n):
    tpw = n // NW
    mesh = plsc.VectorSubcoreMesh(core_axis_name="c", subcore_axis_name="s")
    body = functools.partial(_route_body, tpw=tpw, n=n)
    return pl.kernel(
        body,
        out_type=jax.ShapeDtypeStruct((E * n,), jnp.float32),
        mesh=mesh,
        scratch_types=[
            pltpu.VMEM((E * tpw,), jnp.float32),
            pltpu.VMEM((E * tpw,), jnp.float32),
            pltpu.SemaphoreType.DMA,
        ],
        name="moe_route_sc",
    )(logits_t_flat)


def kernel(x, Wg, bg, W1, W2):
    orig_shape = x.shape
    x_flat = x.reshape(-1, x.shape[-1])
    n = x_flat.shape[0]

    wgt = Wg.T
    w1t = W1.reshape(ER, DIM).T
    w2f = (W2 * SCALING).transpose(0, 2, 1).reshape(ER, DIM)
    bg2 = bg.reshape(1, E)
    expand = jnp.repeat(jnp.eye(E, dtype=jnp.float32), R, axis=1)

    grid = (n // BN,)
    logits_t, act = pl.pallas_call(
        _fwd_body,
        grid=grid,
        in_specs=[
            pl.BlockSpec((BN, DIM), lambda i: (i, 0)),
            pl.BlockSpec((DIM, E), lambda i: (0, 0)),
            pl.BlockSpec((1, E), lambda i: (0, 0)),
            pl.BlockSpec((DIM, ER), lambda i: (0, 0)),
        ],
        out_specs=[
            pl.BlockSpec((E, BN), lambda i: (0, i)),
            pl.BlockSpec((BN, ER), lambda i: (i, 0)),
        ],
        out_shape=[
            jax.ShapeDtypeStruct((E, n), jnp.float32),
            jax.ShapeDtypeStruct((n, ER), jnp.float32),
        ],
        name="moe_fwd_tc",
    )(x_flat, wgt, bg2, w1t)

    gates_t = logits_t

    out = pl.pallas_call(
        _combine_body,
        grid=grid,
        in_specs=[
            pl.BlockSpec((E, BN), lambda i: (0, i)),
            pl.BlockSpec((BN, ER), lambda i: (i, 0)),
            pl.BlockSpec((E, ER), lambda i: (0, 0)),
            pl.BlockSpec((ER, DIM), lambda i: (0, 0)),
        ],
        out_specs=pl.BlockSpec((BN, DIM), lambda i: (i, 0)),
        out_shape=jax.ShapeDtypeStruct((n, DIM), jnp.float32),
        name="moe_combine_tc",
    )(gates_t, act, expand, w2f)

    return out.reshape(orig_shape)
